# Initial kernel scaffold; baseline (speedup 1.0000x reference)
import jax
import jax.numpy as jnp
from jax.experimental import pallas as pl


def kernel(label):
    raise NotImplementedError("write your pallas kernel here")


# speedup vs baseline: 140.5138x; 140.5138x over previous
import jax
import jax.numpy as jnp
from jax.experimental import pallas as pl

N_LABELS_K = 19
H_BLK = 64


def _onehot_body(label_ref, out_ref):
    lab = label_ref[0]
    cls = jax.lax.broadcasted_iota(jnp.int32, (N_LABELS_K, H_BLK, 512), 0)
    out_ref[0] = (lab[None, :, :] == cls).astype(jnp.float32)


def kernel(label):
    N, H, W = label.shape
    grid = (N, H // H_BLK)
    return pl.pallas_call(
        _onehot_body,
        grid=grid,
        in_specs=[pl.BlockSpec((1, H_BLK, W), lambda n, h: (n, h, 0))],
        out_specs=pl.BlockSpec((1, N_LABELS_K, H_BLK, W), lambda n, h: (n, 0, h, 0)),
        out_shape=jax.ShapeDtypeStruct((N, N_LABELS_K, H, W), jnp.float32),
    )(label)


# speedup vs baseline: 181.5288x; 1.2919x over previous
import jax
import jax.numpy as jnp
from jax.experimental import pallas as pl

N_LABELS_K = 19
H_BLK = 128


def _onehot_body(label_ref, out_ref):
    lab = label_ref[0]
    cls = jax.lax.broadcasted_iota(jnp.int32, (N_LABELS_K, H_BLK, 512), 0)
    out_ref[0] = (lab[None, :, :] == cls).astype(jnp.float32)


def kernel(label):
    N, H, W = label.shape
    grid = (N, H // H_BLK)
    return pl.pallas_call(
        _onehot_body,
        grid=grid,
        in_specs=[pl.BlockSpec((1, H_BLK, W), lambda n, h: (n, h, 0))],
        out_specs=pl.BlockSpec((1, N_LABELS_K, H_BLK, W), lambda n, h: (n, 0, h, 0)),
        out_shape=jax.ShapeDtypeStruct((N, N_LABELS_K, H, W), jnp.float32),
    )(label)


# speedup vs baseline: 186.2010x; 1.0257x over previous
import jax
import jax.numpy as jnp
from jax.experimental import pallas as pl

N_LABELS_K = 19
H_BLK = 256


def _onehot_body(label_ref, out_ref):
    lab = label_ref[0]
    cls = jax.lax.broadcasted_iota(jnp.int32, (N_LABELS_K, H_BLK, 512), 0)
    out_ref[0] = (lab[None, :, :] == cls).astype(jnp.float32)


def kernel(label):
    N, H, W = label.shape
    grid = (N, H // H_BLK)
    return pl.pallas_call(
        _onehot_body,
        grid=grid,
        in_specs=[pl.BlockSpec((1, H_BLK, W), lambda n, h: (n, h, 0))],
        out_specs=pl.BlockSpec((1, N_LABELS_K, H_BLK, W), lambda n, h: (n, 0, h, 0)),
        out_shape=jax.ShapeDtypeStruct((N, N_LABELS_K, H, W), jnp.float32),
    )(label)
